# baseline (speedup 1.0000x reference)
import functools

import jax
import jax.numpy as jnp
from jax import lax
from jax.experimental import pallas as pl
from jax.experimental.pallas import tpu as pltpu
from jax.experimental.pallas import tpu_sc as plsc


def _segment_sum_sc(messages, tgt_indices, B, E, D, N):
    NC, NS = 2, 16
    BATCHES_PER_CORE = B // NC
    EPS = 1024
    FULL_LOADS = ((0, 0, 384), (1, 384, 384), (0, 768, 256))
    LAST_LOADS = ((0, 0, 384), (1, 384, 256))
    FULL_CHUNKS = tuple(
        (li, co, 128) for li, nch in ((0, 3), (1, 3), (2, 2)) for co in
        (0, 128, 256)[:nch]
    )
    LAST_CHUNKS = tuple(
        (li, co, 128) for li, nch in ((0, 3), (1, 2)) for co in (0, 128, 256)[:nch]
    )
    NCH = len(FULL_CHUNKS)
    NPAD = ((N + 8 * NS - 1) // (8 * NS)) * (8 * NS)
    ZROWS = NPAD // NS
    ZSUB = 8
    LAST_OROWS = N - (NS - 1) * ZROWS

    mesh = plsc.VectorSubcoreMesh(core_axis_name="c", subcore_axis_name="s")

    @functools.partial(
        pl.kernel,
        out_type=jax.ShapeDtypeStruct((B, N, D), jnp.float32),
        mesh=mesh,
        scratch_types=[
            pltpu.VMEM((384, D), jnp.float32),
            pltpu.VMEM((384, D), jnp.float32),
            *[pltpu.VMEM((1, 128), jnp.int32) for _ in range(NCH)],
            pltpu.VMEM((ZSUB, D), jnp.float32),
            pltpu.VMEM_SHARED((NPAD, D), jnp.float32),
            pltpu.VMEM_SHARED((NPAD, D), jnp.float32),
            pltpu.SemaphoreType.DMA,
            pltpu.SemaphoreType.DMA,
            pltpu.SemaphoreType.DMA,
            pltpu.SemaphoreType.DMA,
            pltpu.SemaphoreType.DMA,
            pltpu.SemaphoreType.DMA,
            pltpu.SemaphoreType.DMA,
        ],
    )
    def k(msg_hbm, tgt_hbm, out_hbm, mv0, mv1, *rest):
        idx_vs = rest[:NCH]
        zeros_v, acc0, acc1, isem, msem0, msem1, ssem, zsem, osem0, osem1 = rest[NCH:]
        msg_vs = (mv0, mv1)
        msems = (msem0, msem1)
        accs = (acc0, acc1)
        osems = (osem0, osem1)
        c = lax.axis_index("c")
        s = lax.axis_index("s")

        def out_copy(b, p, rows):
            r0 = s * ZROWS
            return pltpu.make_async_copy(
                accs[p].at[pl.ds(r0, rows)], out_hbm.at[b, pl.ds(r0, rows)], osems[p]
            )

        def out_start(b, p):
            @pl.when(s < NS - 1)
            def _():
                out_copy(b, p, ZROWS).start()

            @pl.when(s == NS - 1)
            def _():
                out_copy(b, p, LAST_OROWS).start()

        def out_wait(p):
            @pl.when(s < NS - 1)
            def _():
                out_copy(0, p, ZROWS).wait()

            @pl.when(s == NS - 1)
            def _():
                out_copy(0, p, LAST_OROWS).wait()

        @pl.loop(0, ZSUB)
        def _(r):
            @pl.loop(0, D, step=16)
            def _(col):
                zeros_v[r, pl.ds(col, 16)] = jnp.zeros((16,), jnp.float32)

        @pl.loop(0, BATCHES_PER_CORE, step=2)
        def _(bi0):
            for p in range(2):
                bi = bi0 + p
                b = c * BATCHES_PER_CORE + bi
                acc = accs[p]
                ebase = s * EPS

                def msg_copy(loads, li):
                    buf, eoff, rows = loads[li]
                    return pltpu.make_async_copy(
                        msg_hbm.at[b, pl.ds(ebase + eoff, rows)],
                        msg_vs[buf].at[pl.ds(0, rows)],
                        msems[buf],
                    )

                def idx_copy(loads, chunks, j):
                    li, coff, sz = chunks[j]
                    eoff = loads[li][1] + coff
                    return pltpu.make_async_copy(
                        tgt_hbm.at[b, pl.ds(ebase + eoff, sz)], idx_vs[j].at[0], isem
                    )

                def scatter_desc(loads, chunks, j):
                    li, coff, sz = chunks[j]
                    return pltpu.make_async_copy(
                        msg_vs[loads[li][0]].at[pl.ds(coff, sz)],
                        acc.at[idx_vs[j].at[0]],
                        ssem,
                    )

                def prefetch(loads, chunks):
                    msg_copy(loads, 0).start()
                    msg_copy(loads, 1).start()
                    for j in range(len(chunks)):
                        idx_copy(loads, chunks, j).start()

                def scatter_phase(loads, chunks):
                    for j in range(len(chunks)):
                        idx_copy(loads, chunks, j).wait()
                    n0 = sum(1 for ch in chunks if ch[0] == 0)
                    msg_copy(loads, 0).wait()
                    for j in range(n0):
                        scatter_desc(loads, chunks, j).start(add=True)
                    if len(loads) > 2:
                        for j in range(n0):
                            scatter_desc(loads, chunks, j).wait()
                        msg_copy(loads, 2).start()
                    n1 = sum(1 for ch in chunks if ch[0] == 1)
                    msg_copy(loads, 1).wait()
                    for j in range(n0, n0 + n1):
                        scatter_desc(loads, chunks, j).start(add=True)
                    if len(loads) > 2:
                        msg_copy(loads, 2).wait()
                        for j in range(n0 + n1, len(chunks)):
                            scatter_desc(loads, chunks, j).start(add=True)
                        for j in range(n0, len(chunks)):
                            scatter_desc(loads, chunks, j).wait()
                    else:
                        for j in range(len(chunks)):
                            scatter_desc(loads, chunks, j).wait()

                @pl.when(bi >= 2)
                def _():
                    out_wait(p)

                @pl.when(s < NS - 1)
                def _():
                    prefetch(FULL_LOADS, FULL_CHUNKS)

                @pl.when(s == NS - 1)
                def _():
                    prefetch(LAST_LOADS, LAST_CHUNKS)

                for z in range(ZROWS // ZSUB):
                    pltpu.make_async_copy(
                        zeros_v, acc.at[pl.ds(s * ZROWS + z * ZSUB, ZSUB)], zsem
                    ).start()

                for z in range(ZROWS // ZSUB):
                    pltpu.make_async_copy(
                        zeros_v, acc.at[pl.ds(s * ZROWS + z * ZSUB, ZSUB)], zsem
                    ).wait()

                plsc.subcore_barrier()

                @pl.when(s < NS - 1)
                def _():
                    scatter_phase(FULL_LOADS, FULL_CHUNKS)

                @pl.when(s == NS - 1)
                def _():
                    scatter_phase(LAST_LOADS, LAST_CHUNKS)

                plsc.subcore_barrier()

                out_start(b, p)

        out_wait(0)
        out_wait(1)

    return k(messages, tgt_indices)


@jax.jit
def kernel(messages, tgt_indices, atom_features_ref):
    B, E, D = messages.shape
    N = atom_features_ref.shape[1]
    return _segment_sum_sc(messages, tgt_indices, B, E, D, N)
